# Initial kernel scaffold; baseline (speedup 1.0000x reference)
import jax
import jax.numpy as jnp
from jax.experimental import pallas as pl


def kernel(x, position_ids, Wqkv, Wproj, bproj, tk_q, tk_k, tk_v, gamma_freq, theta_freq, phase):
    raise NotImplementedError("write your pallas kernel here")


# speedup vs baseline: 32.2154x; 32.2154x over previous
import math

import jax
import jax.numpy as jnp
from jax import lax
from jax.experimental import pallas as pl
from jax.experimental.pallas import tpu as pltpu

DIM = 1024
HEADS = 16
HD = 64
WIN = 4
SEQ = 2048
KK = 204
PB = 512
RB = 256
NITER = 24
NEG = float(jnp.finfo(jnp.float32).min)
TWO_PI = 2.0 * math.pi


def _prep_body(x_ref, gain_ref, cos_ref, sin_ref, wq_ref, wk_ref, wv_ref,
               tkq_ref, tkk_ref, tkv_ref, q_ref, k_ref, v_ref):
    i = pl.program_id(0)
    base = i * PB
    xe = x_ref[pl.ds(base, PB + 8), :]
    ge = gain_ref[pl.ds(base, PB + 8), :]
    xm = xe * ge

    def conv(t, tk):
        return (tk[0] * t[0:PB] + tk[1] * t[1:PB + 1]
                + tk[2] * t[2:PB + 2] + tk[3] * t[3:PB + 3])

    dn = (((1,), (1,)), ((), ()))
    xmb = xm.astype(jnp.bfloat16)
    q = conv(lax.dot_general(xmb, wq_ref[...].astype(jnp.bfloat16), dn,
                             preferred_element_type=jnp.float32), tkq_ref)
    k = conv(lax.dot_general(xmb, wk_ref[...].astype(jnp.bfloat16), dn,
                             preferred_element_type=jnp.float32), tkk_ref)
    v = conv(lax.dot_general(xmb, wv_ref[...].astype(jnp.bfloat16), dn,
                             preferred_element_type=jnp.float32), tkv_ref)

    cosf = cos_ref[...]
    sinf = sin_ref[...]
    col = lax.broadcasted_iota(jnp.int32, (1, DIM), 1)
    evenm = (col % 2) == 0

    def rope(t):
        sw = jnp.where(evenm, -jnp.roll(t, -1, axis=1), jnp.roll(t, 1, axis=1))
        return t * cosf + sw * sinf

    q_ref[...] = rope(q)
    k_ref[...] = rope(k)
    v_ref[...] = v


def _attn_body(q_ref, k_ref, v_ref, o_ref):
    rb = pl.program_id(1)
    q = q_ref[0]
    k = k_ref[0]
    s = lax.dot_general(q.astype(jnp.bfloat16), k.astype(jnp.bfloat16),
                        (((1,), (1,)), ((), ())),
                        preferred_element_type=jnp.float32) * (HD ** -0.5)
    row = rb * RB + lax.broadcasted_iota(jnp.int32, (RB, SEQ), 0)
    colid = lax.broadcasted_iota(jnp.int32, (RB, SEQ), 1)
    causal = colid <= row
    s = jnp.where(causal, s, NEG)
    rmax = jnp.max(s, axis=1, keepdims=True)
    rmin = jnp.min(jnp.where(causal, s, jnp.inf), axis=1, keepdims=True)
    nvalid = rb * RB + lax.broadcasted_iota(jnp.int32, (RB, 1), 0) + 1

    def body(_, c):
        lo, hi = c
        mid = 0.5 * (lo + hi)
        cnt = jnp.sum((s >= mid).astype(jnp.float32), axis=1, keepdims=True)
        ge = cnt >= KK
        return jnp.where(ge, mid, lo), jnp.where(ge, hi, mid)

    lo, _ = lax.fori_loop(0, NITER, body, (rmin, rmax))
    thr = jnp.where(nvalid <= KK, rmin, lo)
    p = jnp.where(s >= thr, jnp.exp(s - rmax), 0.0)
    probs = p / jnp.sum(p, axis=1, keepdims=True)
    o_ref[0] = lax.dot_general(probs.astype(jnp.bfloat16),
                               v_ref[0].astype(jnp.bfloat16),
                               (((1,), (0,)), ((), ())),
                               preferred_element_type=jnp.float32)


def _proj_body(x_ref, w_ref, b_ref, o_ref):
    o_ref[...] = lax.dot_general(x_ref[...].astype(jnp.bfloat16),
                                 w_ref[...].astype(jnp.bfloat16),
                                 (((1,), (1,)), ((), ())),
                                 preferred_element_type=jnp.float32) + b_ref[...]


def kernel(x, position_ids, Wqkv, Wproj, bproj, tk_q, tk_k, tk_v,
           gamma_freq, theta_freq, phase):
    f32 = jnp.float32
    xs = x[0]
    xp = jnp.pad(xs, ((WIN - 1, 9 - WIN), (0, 0)))
    wq, wk, wv = Wqkv[:DIM], Wqkv[DIM:2 * DIM], Wqkv[2 * DIM:]

    positions = position_ids.astype(f32).reshape(1, -1, 1)
    gamma_phase = (2.0 * math.pi * gamma_freq.reshape(1, 1, -1) * positions
                   / 100.0 + phase.reshape(1, 1, -1))
    theta_phase = 2.0 * math.pi * theta_freq * positions / 100.0
    gain = (0.5 + 0.3 * jnp.cos(gamma_phase)
            + 0.2 * jnp.cos(theta_phase))[0]
    gain_p = jnp.pad(gain, ((WIN - 1, 9 - WIN), (0, 0)))

    inv_freq = 1.0 / (10000.0 ** (jnp.arange(0, HD, 2).astype(f32) / HD))
    freqs = jnp.outer(position_ids.astype(f32), inv_freq)
    cosf = jnp.tile(jnp.repeat(jnp.cos(freqs), 2, axis=1), (1, HEADS))
    sinf = jnp.tile(jnp.repeat(jnp.sin(freqs), 2, axis=1), (1, HEADS))

    q, k, v = pl.pallas_call(
        _prep_body,
        grid=(SEQ // PB,),
        in_specs=[
            pl.BlockSpec((SEQ + 8, DIM), lambda i: (0, 0)),
            pl.BlockSpec((SEQ + 8, DIM), lambda i: (0, 0)),
            pl.BlockSpec((PB, DIM), lambda i: (i, 0)),
            pl.BlockSpec((PB, DIM), lambda i: (i, 0)),
            pl.BlockSpec((DIM, DIM), lambda i: (0, 0)),
            pl.BlockSpec((DIM, DIM), lambda i: (0, 0)),
            pl.BlockSpec((DIM, DIM), lambda i: (0, 0)),
            pl.BlockSpec(memory_space=pltpu.SMEM),
            pl.BlockSpec(memory_space=pltpu.SMEM),
            pl.BlockSpec(memory_space=pltpu.SMEM),
        ],
        out_specs=[pl.BlockSpec((PB, DIM), lambda i: (i, 0))] * 3,
        out_shape=[jax.ShapeDtypeStruct((SEQ, DIM), f32)] * 3,
    )(xp, gain_p, cosf, sinf, wq, wk, wv,
      tk_q.astype(f32), tk_k.astype(f32), tk_v.astype(f32))

    def heads(t):
        return t.reshape(SEQ, HEADS, HD).transpose(1, 0, 2)

    qh, kh, vh = heads(q), heads(k), heads(v)
    ao = pl.pallas_call(
        _attn_body,
        grid=(HEADS, SEQ // RB),
        in_specs=[
            pl.BlockSpec((1, RB, HD), lambda h, r: (h, r, 0)),
            pl.BlockSpec((1, SEQ, HD), lambda h, r: (h, 0, 0)),
            pl.BlockSpec((1, SEQ, HD), lambda h, r: (h, 0, 0)),
        ],
        out_specs=pl.BlockSpec((1, RB, HD), lambda h, r: (h, r, 0)),
        out_shape=jax.ShapeDtypeStruct((HEADS, SEQ, HD), f32),
    )(qh, kh, vh)
    ao = ao.transpose(1, 0, 2).reshape(SEQ, DIM)

    out = pl.pallas_call(
        _proj_body,
        grid=(SEQ // PB,),
        in_specs=[
            pl.BlockSpec((PB, DIM), lambda i: (i, 0)),
            pl.BlockSpec((DIM, DIM), lambda i: (0, 0)),
            pl.BlockSpec((1, DIM), lambda i: (0, 0)),
        ],
        out_specs=pl.BlockSpec((PB, DIM), lambda i: (i, 0)),
        out_shape=jax.ShapeDtypeStruct((SEQ, DIM), f32),
    )(ao, Wproj, bproj.reshape(1, DIM))
    return out.reshape(1, SEQ, DIM)
